# Initial kernel scaffold; baseline (speedup 1.0000x reference)
import jax
import jax.numpy as jnp
from jax.experimental import pallas as pl


def kernel(x, edge_index, Wl0, bl0, Wr0, Wl1, bl1, Wr1, Wl2, bl2, Wr2, ln_g, ln_b):
    raise NotImplementedError("write your pallas kernel here")


# speedup vs baseline: 5.2404x; 5.2404x over previous
import functools

import jax
import jax.numpy as jnp
from jax import lax
from jax.experimental import pallas as pl
from jax.experimental.pallas import tpu as pltpu
from jax.experimental.pallas import tpu_sc as plsc

N = 10000
NP = 10240
E = 320000
D = 128
EPS = 1e-5

NC = 2
NS = 16
NW = NC * NS
EW = E // NW
CHUNK = 80
ITERS = EW // CHUNK
RPT = NP // NS
ZR = 32
CNT_CHUNK = 2048


def _sc_agg_body(with_cnt, *refs):
    if with_cnt:
        (y_hbm, src_hbm, dst_hbm, agg_out, cnt_out,
         src_v, dst_v, rows_v, zrow_v, ones_v, zcnt_v, acc_sh, cnt_sh, sem) = refs
    else:
        (y_hbm, src_hbm, dst_hbm, agg_out,
         src_v, dst_v, rows_v, zrow_v, acc_sh, sem) = refs

    cid = lax.axis_index("c")
    sid = lax.axis_index("s")
    wid = sid * NC + cid

    zvec = jnp.zeros((16,), jnp.float32)
    for r in range(ZR):
        for j in range(D // 16):
            zrow_v[r, pl.ds(j * 16, 16)] = zvec

    def zero_body(g, _):
        pltpu.sync_copy(zrow_v, acc_sh.at[pl.ds(sid * RPT + g * ZR, ZR)])
        return 0
    lax.fori_loop(0, RPT // ZR, zero_body, 0)

    if with_cnt:
        for j in range(CHUNK // 16):
            ones_v[pl.ds(j * 16, 16)] = jnp.ones((16,), jnp.float32)
        for j in range(CNT_CHUNK // 16):
            zcnt_v[pl.ds(j * 16, 16)] = zvec

        @pl.when(sid == 0)
        def _():
            def zero_cnt(g, _):
                pltpu.sync_copy(zcnt_v, cnt_sh.at[pl.ds(g * CNT_CHUNK, CNT_CHUNK)])
                return 0
            lax.fori_loop(0, NP // CNT_CHUNK, zero_cnt, 0)

    plsc.subcore_barrier()

    base = wid * EW

    def edge_body(g, _):
        off = base + g * CHUNK
        pltpu.sync_copy(src_hbm.at[pl.ds(off, CHUNK)], src_v)
        pltpu.sync_copy(dst_hbm.at[pl.ds(off, CHUNK)], dst_v)
        pltpu.async_copy(y_hbm.at[src_v], rows_v, sem).wait()
        pltpu.sync_copy(rows_v, acc_sh.at[dst_v], add=True)
        if with_cnt:
            pltpu.sync_copy(ones_v, cnt_sh.at[dst_v], add=True)
        return 0
    lax.fori_loop(0, ITERS, edge_body, 0)

    plsc.subcore_barrier()

    pltpu.sync_copy(acc_sh.at[pl.ds(sid * RPT, RPT)],
                    agg_out.at[cid, pl.ds(sid * RPT, RPT)])
    if with_cnt:
        pltpu.sync_copy(cnt_sh.at[pl.ds(sid * RPT, RPT)],
                        cnt_out.at[cid, pl.ds(sid * RPT, RPT)])


@functools.lru_cache(maxsize=None)
def _make_sc_agg(with_cnt):
    mesh = plsc.VectorSubcoreMesh(core_axis_name="c", subcore_axis_name="s",
                                  num_cores=NC, num_subcores=NS)
    out_type = [jax.ShapeDtypeStruct((NC, NP, D), jnp.float32)]
    scratch = [
        pltpu.VMEM((CHUNK,), jnp.int32),
        pltpu.VMEM((CHUNK,), jnp.int32),
        pltpu.VMEM((CHUNK, D), jnp.float32),
        pltpu.VMEM((ZR, D), jnp.float32),
    ]
    if with_cnt:
        out_type.append(jax.ShapeDtypeStruct((NC, NP), jnp.float32))
        scratch += [
            pltpu.VMEM((CHUNK,), jnp.float32),
            pltpu.VMEM((CNT_CHUNK,), jnp.float32),
        ]
    scratch.append(pltpu.VMEM_SHARED((NP, D), jnp.float32))
    if with_cnt:
        scratch.append(pltpu.VMEM_SHARED((NP,), jnp.float32))
    scratch.append(pltpu.SemaphoreType.DMA)

    return pl.kernel(
        functools.partial(_sc_agg_body, with_cnt),
        out_type=out_type,
        mesh=mesh,
        scratch_types=scratch,
    )


R = 1024
_G = NP // R


def _tc_pre_body(x_ref, wl_ref, wr_ref, bl_ref, y_out, z_out):
    h = x_ref[...]
    y_out[...] = jnp.dot(h, wl_ref[...], preferred_element_type=jnp.float32)
    z_out[...] = jnp.dot(h, wr_ref[...], preferred_element_type=jnp.float32) + bl_ref[...]


def _combine_mean(agg_ref, cnt_ref, z_ref):
    i = pl.program_id(0)
    s = agg_ref[0] + agg_ref[1]
    c = cnt_ref[0, pl.ds(i * R, R)] + cnt_ref[1, pl.ds(i * R, R)]
    inv = 1.0 / jnp.clip(c, 1.0, None)
    return s * inv[:, None] + z_ref[...]


def _tc_mid_body(agg_ref, cnt_ref, z_ref, wl_ref, wr_ref, bl_ref, g_ref, b_ref,
                 y_out, z_out):
    pre = _combine_mean(agg_ref, cnt_ref, z_ref)
    h = jnp.maximum(pre, 0.0)
    mu = jnp.mean(h, axis=-1, keepdims=True)
    var = jnp.mean((h - mu) ** 2, axis=-1, keepdims=True)
    hn = (h - mu) * lax.rsqrt(var + EPS) * g_ref[...] + b_ref[...]
    y_out[...] = jnp.dot(hn, wl_ref[...], preferred_element_type=jnp.float32)
    z_out[...] = jnp.dot(hn, wr_ref[...], preferred_element_type=jnp.float32) + bl_ref[...]


def _tc_post_body(agg_ref, cnt_ref, z_ref, out_ref):
    out_ref[...] = _combine_mean(agg_ref, cnt_ref, z_ref)


_row_spec = pl.BlockSpec((R, D), lambda i: (i, 0))
_w_spec = pl.BlockSpec((D, D), lambda i: (0, 0))
_b_spec = pl.BlockSpec((1, D), lambda i: (0, 0))
_agg_spec = pl.BlockSpec((NC, R, D), lambda i: (0, i, 0))
_cnt_spec = pl.BlockSpec((NC, NP), lambda i: (0, 0))

_tc_pre = pl.pallas_call(
    _tc_pre_body,
    grid=(_G,),
    in_specs=[_row_spec, _w_spec, _w_spec, _b_spec],
    out_specs=[_row_spec, _row_spec],
    out_shape=[jax.ShapeDtypeStruct((NP, D), jnp.float32)] * 2,
)

_tc_mid = pl.pallas_call(
    _tc_mid_body,
    grid=(_G,),
    in_specs=[_agg_spec, _cnt_spec, _row_spec, _w_spec, _w_spec,
              _b_spec, _b_spec, _b_spec],
    out_specs=[_row_spec, _row_spec],
    out_shape=[jax.ShapeDtypeStruct((NP, D), jnp.float32)] * 2,
)

_tc_post = pl.pallas_call(
    _tc_post_body,
    grid=(_G,),
    in_specs=[_agg_spec, _cnt_spec, _row_spec],
    out_specs=_row_spec,
    out_shape=jax.ShapeDtypeStruct((NP, D), jnp.float32),
)


def kernel(x, edge_index, Wl0, bl0, Wr0, Wl1, bl1, Wr1, Wl2, bl2, Wr2, ln_g, ln_b):
    src = edge_index[0]
    dst = edge_index[1]
    xp = jnp.pad(x, ((0, NP - N), (0, 0)))
    bl0_2 = bl0.reshape(1, D)
    bl1_2 = bl1.reshape(1, D)
    bl2_2 = bl2.reshape(1, D)
    g2 = ln_g.reshape(1, D)
    b2 = ln_b.reshape(1, D)

    sc_agg_cnt = _make_sc_agg(True)
    sc_agg = _make_sc_agg(False)

    y0, z0 = _tc_pre(xp, Wl0.T, Wr0.T, bl0_2)
    agg0, cnt = sc_agg_cnt(y0, src, dst)
    y1, z1 = _tc_mid(agg0, cnt, z0, Wl1.T, Wr1.T, bl1_2, g2, b2)
    (agg1,) = sc_agg(y1, src, dst)
    y2, z2 = _tc_mid(agg1, cnt, z1, Wl2.T, Wr2.T, bl2_2, g2, b2)
    (agg2,) = sc_agg(y2, src, dst)
    out = _tc_post(agg2, cnt, z2)
    return out[:N]
